# Initial kernel scaffold; baseline (speedup 1.0000x reference)
import jax
import jax.numpy as jnp
from jax.experimental import pallas as pl


def kernel(node_embed, edge_dist, edge_index, W_res, W_msg, b_msg, W_upd, b_upd):
    raise NotImplementedError("write your pallas kernel here")


# speedup vs baseline: 2.9999x; 2.9999x over previous
import functools

import jax
import jax.numpy as jnp
from jax import lax
from jax.experimental import pallas as pl
from jax.experimental.pallas import tpu as pltpu
from jax.experimental.pallas import tpu_sc as plsc

N_NODES = 10000
N_EDGES = 320000
F = 128

NC = 2
NS = 16
L = 16
NW = NC * NS
E_PER_W = N_EDGES // NW
CHUNK = 80
N_CHUNKS = E_PER_W // CHUNK
ZROWS = 200
N_ZCHUNKS = N_NODES // ZROWS
ZPT = 4

_TC_BLOCK = 2000


def _proj_body(x_ref, wsT_ref, wdT_ref, b_ref, p_ref, q_ref):
    x = x_ref[...]
    p_ref[...] = jnp.dot(x, wsT_ref[...], preferred_element_type=jnp.float32)
    q_ref[...] = (
        jnp.dot(x, wdT_ref[...], preferred_element_type=jnp.float32) + b_ref[...]
    )


def _project(x, wsT, wdT, b):
    grid = (N_NODES // _TC_BLOCK,)
    blk = pl.BlockSpec((_TC_BLOCK, F), lambda i: (i, 0))
    wblk = pl.BlockSpec((F, F), lambda i: (0, 0))
    bblk = pl.BlockSpec((1, F), lambda i: (0, 0))
    return pl.pallas_call(
        _proj_body,
        grid=grid,
        in_specs=[blk, wblk, wblk, bblk],
        out_specs=[blk, blk],
        out_shape=[
            jax.ShapeDtypeStruct((N_NODES, F), jnp.float32),
            jax.ShapeDtypeStruct((N_NODES, F), jnp.float32),
        ],
    )(x, wsT, wdT, b)


def _edge_body(row_hbm, col_hbm, dist_hbm, p_hbm, q_hbm, wd_hbm, zero_hbm,
               out_hbm,
               ridx_v, cidx_v, dist_v, p_v, q_v, wd_v, zbuf_v, aggr_sh,
               sem1, sem2):
    cid = lax.axis_index("c")
    sid = lax.axis_index("s")
    wid = cid * NS + sid

    pltpu.sync_copy(wd_hbm, wd_v)

    pltpu.sync_copy(zero_hbm, zbuf_v)
    for j in range(ZPT):
        c = sid * ZPT + j

        @pl.when(c < N_ZCHUNKS)
        def _():
            pltpu.sync_copy(zbuf_v, aggr_sh.at[pl.ds(c * ZROWS, ZROWS)])

    plsc.subcore_barrier()

    def chunk_body(i, carry):
        base = wid * E_PER_W + i * CHUNK
        pltpu.sync_copy(row_hbm.at[pl.ds(base, CHUNK)], ridx_v)
        pltpu.sync_copy(col_hbm.at[pl.ds(base, CHUNK)], cidx_v)
        pltpu.sync_copy(dist_hbm.at[pl.ds(base, CHUNK)], dist_v)
        cp1 = pltpu.async_copy(p_hbm.at[ridx_v], p_v, sem1)
        cp2 = pltpu.async_copy(q_hbm.at[cidx_v], q_v, sem2)
        cp1.wait()
        cp2.wait()

        def group_body(g, c):
            dvec = dist_v[pl.ds(g * L, L)]
            for j in range(L):
                e = g * L + j
                dsplat = jnp.full((L,), dvec[j], jnp.float32)
                for k in range(F // L):
                    s = pl.ds(k * L, L)
                    m = p_v[e, s] + q_v[e, s] + dsplat * wd_v[s]
                    p_v[e, s] = jnp.maximum(m, 0.0)
            return c

        lax.fori_loop(0, CHUNK // L, group_body, 0, unroll=False)
        pltpu.sync_copy(p_v, aggr_sh.at[cidx_v], add=True)
        return carry

    lax.fori_loop(0, N_CHUNKS, chunk_body, 0, unroll=False)
    plsc.subcore_barrier()

    for j in range(ZPT):
        c = sid * ZPT + j

        @pl.when(c < N_ZCHUNKS)
        def _():
            off = c * ZROWS
            pltpu.sync_copy(aggr_sh.at[pl.ds(off, ZROWS)], zbuf_v)
            pltpu.sync_copy(zbuf_v, out_hbm.at[cid, pl.ds(off, ZROWS)])


def _edge_aggregate(row, col, dist, p, q, wd, zero_rows):
    mesh = plsc.VectorSubcoreMesh(
        core_axis_name="c", subcore_axis_name="s", num_cores=NC, num_subcores=NS
    )
    kern = pl.kernel(
        _edge_body,
        out_type=jax.ShapeDtypeStruct((NC, N_NODES, F), jnp.float32),
        mesh=mesh,
        scratch_types=[
            pltpu.VMEM((CHUNK,), jnp.int32),
            pltpu.VMEM((CHUNK,), jnp.int32),
            pltpu.VMEM((CHUNK,), jnp.float32),
            pltpu.VMEM((CHUNK, F), jnp.float32),
            pltpu.VMEM((CHUNK, F), jnp.float32),
            pltpu.VMEM((F,), jnp.float32),
            pltpu.VMEM((ZROWS, F), jnp.float32),
            pltpu.VMEM_SHARED((N_NODES, F), jnp.float32),
            pltpu.SemaphoreType.DMA,
            pltpu.SemaphoreType.DMA,
        ],
    )
    return kern(row, col, dist, p, q, wd, zero_rows)


def _upd_body(x_ref, a0_ref, a1_ref, wrT_ref, waT_ref, wbT_ref, b_ref, o_ref):
    x = x_ref[...]
    a = a0_ref[...] + a1_ref[...]
    h = (
        jnp.dot(x, waT_ref[...], preferred_element_type=jnp.float32)
        + jnp.dot(a, wbT_ref[...], preferred_element_type=jnp.float32)
        + b_ref[...]
    )
    o_ref[...] = jnp.dot(x, wrT_ref[...], preferred_element_type=jnp.float32) + (
        jnp.maximum(h, 0.0)
    )


def _update(x, a0, a1, wrT, waT, wbT, b):
    grid = (N_NODES // _TC_BLOCK,)
    blk = pl.BlockSpec((_TC_BLOCK, F), lambda i: (i, 0))
    wblk = pl.BlockSpec((F, F), lambda i: (0, 0))
    bblk = pl.BlockSpec((1, F), lambda i: (0, 0))
    return pl.pallas_call(
        _upd_body,
        grid=grid,
        in_specs=[blk, blk, blk, wblk, wblk, wblk, bblk],
        out_specs=blk,
        out_shape=jax.ShapeDtypeStruct((N_NODES, F), jnp.float32),
    )(x, a0, a1, wrT, waT, wbT, b)


def kernel(node_embed, edge_dist, edge_index, W_res, W_msg, b_msg, W_upd, b_upd):
    row = edge_index[0].astype(jnp.int32)
    col = edge_index[1].astype(jnp.int32)
    dist = edge_dist.reshape(-1)

    wsT = W_msg[:, :F].T
    wdT = W_msg[:, F : 2 * F].T
    wd = W_msg[:, 2 * F]
    b2 = b_msg.reshape(1, F)

    p, q = _project(node_embed, wsT, wdT, b2)

    zero_rows = jnp.zeros((ZROWS, F), jnp.float32)
    partials = _edge_aggregate(row, col, dist, p, q, wd, zero_rows)

    wrT = W_res.T
    waT = W_upd[:, :F].T
    wbT = W_upd[:, F:].T
    return _update(
        node_embed, partials[0], partials[1], wrT, waT, wbT, b_upd.reshape(1, F)
    )
